# baseline (speedup 1.0000x reference)
import functools

import jax
import jax.numpy as jnp
from jax import lax
from jax.experimental import pallas as pl
from jax.experimental.pallas import tpu as pltpu
from jax.experimental.pallas import tpu_sc as plsc

N = 10000
E = 160000
D = 256
R = 256
P = 64

_NC = 2
_NS = 16
_NW = _NC * _NS
_EPW = E // _NW
_C = 128
_NFULL = _EPW // _C
_CT = _EPW - _NFULL * _C
_EH = E // 2

_HI = -65536
_LO = 65535
_RND = 0x8000


def _pack_pair(q32, k32):
    qp = (q32 + _RND) & _HI
    kp = lax.shift_right_logical(k32 + _RND, 16)
    return qp | kp


def _node_body(nf_ref, wn_ref, bn_ref, wsrc_ref, wdst_ref, tbl_ref):
    n = jnp.maximum(
        jnp.dot(nf_ref[...], wn_ref[...], preferred_element_type=jnp.float32)
        + bn_ref[...], 0.0)
    q = jnp.dot(n, wsrc_ref[...], preferred_element_type=jnp.float32)
    k = jnp.dot(n, wdst_ref[...], preferred_element_type=jnp.float32)
    tbl_ref[...] = _pack_pair(lax.bitcast_convert_type(q, jnp.int32),
                              lax.bitcast_convert_type(k, jnp.int32))


def _node_tc(nf, Wn, bn, Wsrc, Wdst):
    blk = 2000
    return pl.pallas_call(
        _node_body,
        grid=(N // blk,),
        in_specs=[
            pl.BlockSpec((blk, D), lambda i: (i, 0)),
            pl.BlockSpec((D, R), lambda i: (0, 0)),
            pl.BlockSpec((1, R), lambda i: (0, 0)),
            pl.BlockSpec((R, P), lambda i: (0, 0)),
            pl.BlockSpec((R, P), lambda i: (0, 0)),
        ],
        out_specs=pl.BlockSpec((blk, P), lambda i: (i, 0)),
        out_shape=jax.ShapeDtypeStruct((N, P), jnp.int32),
    )(nf, Wn, bn.reshape(1, R), Wsrc, Wdst)


def _sc_gather_body(tbl_hbm, src_hbm, dst_hbm, out_hbm,
                    sidx_all, didx_all,
                    a_a, b_a, p_a, a_b, b_b, p_b,
                    ta_v, tb_v, tp_v,
                    sg_a, sg_b, sw_a, sw_b, sem_t):
    wid = lax.axis_index("s") * _NC + lax.axis_index("c")
    base_w = pl.multiple_of(wid * _EPW, 8)
    row_w = pl.multiple_of((wid % (_NW // 2)) * _EPW, 8)
    col_w = pl.multiple_of((wid // (_NW // 2)) * P, P)

    pltpu.sync_copy(src_hbm.at[pl.ds(base_w, _EPW)], sidx_all)
    pltpu.sync_copy(dst_hbm.at[pl.ds(base_w, _EPW)], didx_all)

    def issue_gather(c, av, bv, sem):
        off = pl.multiple_of(c * _C, 8)
        ca = pltpu.async_copy(tbl_hbm.at[sidx_all.at[pl.ds(off, _C)]], av, sem)
        cb = pltpu.async_copy(tbl_hbm.at[didx_all.at[pl.ds(off, _C)]], bv, sem)
        return ca, cb

    def pack(c_rows, av, bv, pv):
        def row_body(i, _):
            for s in range(P // 16):
                sl = (i, pl.ds(s * 16, 16))
                pv[sl] = (av[sl] & _HI) | (bv[sl] & _LO)
            return 0
        lax.fori_loop(0, c_rows, row_body, 0, unroll=4)

    def issue_writeback(c, pv, sem):
        row = pl.multiple_of(row_w + c * _C, 8)
        return pltpu.async_copy(
            pv, out_hbm.at[pl.ds(row, _C), pl.ds(col_w, P)], sem)

    def pair_body(i, _):
        c0 = 2 * i
        ga = issue_gather(c0, a_a, b_a, sg_a)
        gb = issue_gather(c0 + 1, a_b, b_b, sg_b)
        ga[0].wait()
        ga[1].wait()
        pack(_C, a_a, b_a, p_a)
        wa = issue_writeback(c0, p_a, sw_a)
        gb[0].wait()
        gb[1].wait()
        pack(_C, a_b, b_b, p_b)
        wb = issue_writeback(c0 + 1, p_b, sw_b)
        wa.wait()
        wb.wait()
        return 0

    lax.fori_loop(0, _NFULL // 2, pair_body, 0)

    ga = issue_gather(_NFULL - 1, a_a, b_a, sg_a)
    toff = pl.multiple_of(_NFULL * _C, 8)
    ca = pltpu.async_copy(tbl_hbm.at[sidx_all.at[pl.ds(toff, _CT)]],
                          ta_v, sem_t)
    cb = pltpu.async_copy(tbl_hbm.at[didx_all.at[pl.ds(toff, _CT)]],
                          tb_v, sem_t)
    ga[0].wait()
    ga[1].wait()
    pack(_C, a_a, b_a, p_a)
    wa = issue_writeback(_NFULL - 1, p_a, sw_a)
    ca.wait()
    cb.wait()
    pack(_CT, ta_v, tb_v, tp_v)
    trow = pl.multiple_of(row_w + _NFULL * _C, 8)
    pltpu.sync_copy(tp_v, out_hbm.at[pl.ds(trow, _CT), pl.ds(col_w, P)])
    wa.wait()


def _sc_gather(tbl, src, dst):
    mesh = plsc.VectorSubcoreMesh(core_axis_name="c", subcore_axis_name="s")
    kern = functools.partial(
        pl.kernel,
        out_type=jax.ShapeDtypeStruct((_EH, 2 * P), jnp.int32),
        mesh=mesh,
        scratch_types=[
            pltpu.VMEM((_EPW,), jnp.int32),
            pltpu.VMEM((_EPW,), jnp.int32),
            pltpu.VMEM((_C, P), jnp.int32),
            pltpu.VMEM((_C, P), jnp.int32),
            pltpu.VMEM((_C, P), jnp.int32),
            pltpu.VMEM((_C, P), jnp.int32),
            pltpu.VMEM((_C, P), jnp.int32),
            pltpu.VMEM((_C, P), jnp.int32),
            pltpu.VMEM((_CT, P), jnp.int32),
            pltpu.VMEM((_CT, P), jnp.int32),
            pltpu.VMEM((_CT, P), jnp.int32),
            pltpu.SemaphoreType.DMA,
            pltpu.SemaphoreType.DMA,
            pltpu.SemaphoreType.DMA,
            pltpu.SemaphoreType.DMA,
            pltpu.SemaphoreType.DMA,
        ],
        compiler_params=pltpu.CompilerParams(use_tc_tiling_on_sc=False),
    )(_sc_gather_body)
    return kern(tbl, src, dst)


_EBLK = 3200
_NSTEP = _EH // _EBLK


def _edge_chain(ef, we, be, wedge, x_bits):
    e = jnp.maximum(
        jnp.dot(ef, we, preferred_element_type=jnp.float32) + be, 0.0)
    ep = jnp.dot(e, wedge, preferred_element_type=jnp.float32)
    qs = lax.bitcast_convert_type(x_bits & _HI, jnp.float32)
    kd = lax.bitcast_convert_type(lax.shift_left(x_bits, 16), jnp.float32)
    s = jnp.sum(qs * kd + ep * (qs + kd), axis=-1)
    return s.reshape(1, _EBLK // 128, 128)


def _edge_body(efa_ref, efb_ref, we_ref, be_ref, wedge_ref, qskd_ref,
               sa_ref, sb_ref):
    we = we_ref[...]
    be = be_ref[...]
    wedge = wedge_ref[...]
    x = qskd_ref[...]
    sa_ref[...] = _edge_chain(efa_ref[...], we, be, wedge, x[:, :P])
    sb_ref[...] = _edge_chain(efb_ref[...], we, be, wedge, x[:, P:])


def _edge_tc(ef, We, be, Wedge, qskd):
    nrow = _EBLK // 128
    sa, sb = pl.pallas_call(
        _edge_body,
        grid=(_NSTEP,),
        in_specs=[
            pl.BlockSpec((_EBLK, D), lambda i: (i, 0)),
            pl.BlockSpec((_EBLK, D), lambda i: (i + _NSTEP, 0)),
            pl.BlockSpec((D, R), lambda i: (0, 0)),
            pl.BlockSpec((1, R), lambda i: (0, 0)),
            pl.BlockSpec((R, P), lambda i: (0, 0)),
            pl.BlockSpec((_EBLK, 2 * P), lambda i: (i, 0)),
        ],
        out_specs=[
            pl.BlockSpec((1, nrow, 128), lambda i: (i, 0, 0)),
            pl.BlockSpec((1, nrow, 128), lambda i: (i, 0, 0)),
        ],
        out_shape=[jax.ShapeDtypeStruct((_NSTEP, nrow, 128), jnp.float32)] * 2,
    )(ef, ef, We, be.reshape(1, R), Wedge, qskd)
    return jnp.concatenate([sa.reshape(_EH), sb.reshape(_EH)])


def kernel(node_features, edge_features, edge_index, Wn, bn, We, be,
           Wsrc, Wdst, Wedge):
    tbl = _node_tc(node_features, Wn, bn, Wsrc, Wdst)
    src = edge_index[0].astype(jnp.int32)
    dst = edge_index[1].astype(jnp.int32)
    qskd = _sc_gather(tbl, src, dst)
    return _edge_tc(edge_features, We, be, Wedge, qskd)


# speedup vs baseline: 1.8830x; 1.8830x over previous
import functools

import jax
import jax.numpy as jnp
from jax import lax
from jax.experimental import pallas as pl
from jax.experimental.pallas import tpu as pltpu
from jax.experimental.pallas import tpu_sc as plsc

N = 10000
E = 160000
D = 256
R = 256
P = 64

_NC = 2
_NS = 16
_NW = _NC * _NS
_EPW = E // _NW
_C = 128
_NFULL = _EPW // _C
_CT = _EPW - _NFULL * _C


def _node_body(nf_ref, wn_ref, bn_ref, wsrc_ref, wdst_ref, qk_ref):
    n = jnp.maximum(
        jnp.dot(nf_ref[...], wn_ref[...], preferred_element_type=jnp.float32)
        + bn_ref[...], 0.0)
    q = jnp.dot(n, wsrc_ref[...], preferred_element_type=jnp.float32)
    k = jnp.dot(n, wdst_ref[...], preferred_element_type=jnp.float32)
    qk_ref[...] = jnp.concatenate([q, k], axis=-1)


def _node_tc(nf, Wn, bn, Wsrc, Wdst):
    blk = 2000
    return pl.pallas_call(
        _node_body,
        grid=(N // blk,),
        in_specs=[
            pl.BlockSpec((blk, D), lambda i: (i, 0)),
            pl.BlockSpec((D, R), lambda i: (0, 0)),
            pl.BlockSpec((1, R), lambda i: (0, 0)),
            pl.BlockSpec((R, P), lambda i: (0, 0)),
            pl.BlockSpec((R, P), lambda i: (0, 0)),
        ],
        out_specs=pl.BlockSpec((blk, 2 * P), lambda i: (i, 0)),
        out_shape=jax.ShapeDtypeStruct((N, 2 * P), jnp.float32),
    )(nf, Wn, bn.reshape(1, R), Wsrc, Wdst)


def _sc_gather_body(tbl_hbm, src2_hbm, dst2_hbm, qskd_hbm,
                    sidx_all, didx_all,
                    qs_a, kd_a, qs_b, kd_b,
                    tqs_v, tkd_v,
                    sg_a, sg_b, sw_a, sw_b, sem_t):
    wid = lax.axis_index("s") * _NC + lax.axis_index("c")
    base_w = pl.multiple_of(wid * _EPW, 8)

    pltpu.sync_copy(src2_hbm.at[pl.ds(base_w, _EPW)], sidx_all)
    pltpu.sync_copy(dst2_hbm.at[pl.ds(base_w, _EPW)], didx_all)

    def issue_gather(c, qs, kd, sem):
        off = pl.multiple_of(c * _C, 8)
        cq = pltpu.async_copy(tbl_hbm.at[sidx_all.at[pl.ds(off, _C)]], qs, sem)
        ck = pltpu.async_copy(tbl_hbm.at[didx_all.at[pl.ds(off, _C)]], kd, sem)
        return cq, ck

    def issue_writeback(c, qs, kd, sem):
        base = pl.multiple_of(base_w + c * _C, 8)
        wq = pltpu.async_copy(
            qs, qskd_hbm.at[pl.ds(base, _C), pl.ds(0, P)], sem)
        wk = pltpu.async_copy(
            kd, qskd_hbm.at[pl.ds(base, _C), pl.ds(P, P)], sem)
        return wq, wk

    def pair_body(i, _):
        c0 = 2 * i
        ga = issue_gather(c0, qs_a, kd_a, sg_a)
        gb = issue_gather(c0 + 1, qs_b, kd_b, sg_b)
        ga[0].wait()
        ga[1].wait()
        wa = issue_writeback(c0, qs_a, kd_a, sw_a)
        gb[0].wait()
        gb[1].wait()
        wb = issue_writeback(c0 + 1, qs_b, kd_b, sw_b)
        wa[0].wait()
        wa[1].wait()
        wb[0].wait()
        wb[1].wait()
        return 0

    lax.fori_loop(0, _NFULL // 2, pair_body, 0)

    ga = issue_gather(_NFULL - 1, qs_a, kd_a, sg_a)
    toff = pl.multiple_of(_NFULL * _C, 8)
    tbase = pl.multiple_of(base_w + _NFULL * _C, 8)
    cq = pltpu.async_copy(tbl_hbm.at[sidx_all.at[pl.ds(toff, _CT)]],
                          tqs_v, sem_t)
    ck = pltpu.async_copy(tbl_hbm.at[didx_all.at[pl.ds(toff, _CT)]],
                          tkd_v, sem_t)
    ga[0].wait()
    ga[1].wait()
    wa = issue_writeback(_NFULL - 1, qs_a, kd_a, sw_a)
    cq.wait()
    ck.wait()
    pltpu.sync_copy(tqs_v, qskd_hbm.at[pl.ds(tbase, _CT), pl.ds(0, P)])
    pltpu.sync_copy(tkd_v, qskd_hbm.at[pl.ds(tbase, _CT), pl.ds(P, P)])
    wa[0].wait()
    wa[1].wait()


def _sc_gather(qk_tbl, src2, dst2):
    mesh = plsc.VectorSubcoreMesh(core_axis_name="c", subcore_axis_name="s")
    kern = functools.partial(
        pl.kernel,
        out_type=jax.ShapeDtypeStruct((E, 2 * P), jnp.float32),
        mesh=mesh,
        scratch_types=[
            pltpu.VMEM((_EPW,), jnp.int32),
            pltpu.VMEM((_EPW,), jnp.int32),
            pltpu.VMEM((_C, P), jnp.float32),
            pltpu.VMEM((_C, P), jnp.float32),
            pltpu.VMEM((_C, P), jnp.float32),
            pltpu.VMEM((_C, P), jnp.float32),
            pltpu.VMEM((_CT, P), jnp.float32),
            pltpu.VMEM((_CT, P), jnp.float32),
            pltpu.SemaphoreType.DMA,
            pltpu.SemaphoreType.DMA,
            pltpu.SemaphoreType.DMA,
            pltpu.SemaphoreType.DMA,
            pltpu.SemaphoreType.DMA,
        ],
        compiler_params=pltpu.CompilerParams(use_tc_tiling_on_sc=False),
    )(_sc_gather_body)
    return kern(qk_tbl, src2, dst2)


_EBLK = 3200


def _edge_body(ef_ref, we_ref, be_ref, wedge_ref, qskd_ref, score_ref):
    e = jnp.maximum(
        jnp.dot(ef_ref[...], we_ref[...], preferred_element_type=jnp.float32)
        + be_ref[...], 0.0)
    ep = jnp.dot(e, wedge_ref[...], preferred_element_type=jnp.float32)
    qs = qskd_ref[:, :P]
    kd = qskd_ref[:, P:]
    s = jnp.sum(qs * kd + ep * (qs + kd), axis=-1)
    score_ref[...] = s.reshape(1, _EBLK // 128, 128)


def _edge_tc(ef, We, be, Wedge, qskd):
    out = pl.pallas_call(
        _edge_body,
        grid=(E // _EBLK,),
        in_specs=[
            pl.BlockSpec((_EBLK, D), lambda i: (i, 0)),
            pl.BlockSpec((D, R), lambda i: (0, 0)),
            pl.BlockSpec((1, R), lambda i: (0, 0)),
            pl.BlockSpec((R, P), lambda i: (0, 0)),
            pl.BlockSpec((_EBLK, 2 * P), lambda i: (i, 0)),
        ],
        out_specs=pl.BlockSpec((1, _EBLK // 128, 128), lambda i: (i, 0, 0)),
        out_shape=jax.ShapeDtypeStruct(
            (E // _EBLK, _EBLK // 128, 128), jnp.float32),
    )(ef, We, be.reshape(1, R), Wedge, qskd)
    return out.reshape(E)


def kernel(node_features, edge_features, edge_index, Wn, bn, We, be,
           Wsrc, Wdst, Wedge):
    qk = _node_tc(node_features, Wn, bn, Wsrc, Wdst)
    qk_tbl = qk.reshape(2 * N, P)
    src2 = edge_index[0].astype(jnp.int32) * 2
    dst2 = edge_index[1].astype(jnp.int32) * 2 + 1
    qskd = _sc_gather(qk_tbl, src2, dst2)
    return _edge_tc(edge_features, We, be, Wedge, qskd)


# speedup vs baseline: 1.9158x; 1.0175x over previous
import functools

import jax
import jax.numpy as jnp
from jax import lax
from jax.experimental import pallas as pl
from jax.experimental.pallas import tpu as pltpu
from jax.experimental.pallas import tpu_sc as plsc

N = 10000
E = 160000
D = 256
R = 256
P = 64

_NC = 2
_NS = 16
_NW = _NC * _NS
_EPW = E // _NW
_C = 128
_NFULL = _EPW // _C
_CT = _EPW - _NFULL * _C


def _node_body(nf_ref, wn_ref, bn_ref, wsrc_ref, wdst_ref, qk_ref):
    n = jnp.maximum(
        jnp.dot(nf_ref[...], wn_ref[...], preferred_element_type=jnp.float32)
        + bn_ref[...], 0.0)
    q = jnp.dot(n, wsrc_ref[...], preferred_element_type=jnp.float32)
    k = jnp.dot(n, wdst_ref[...], preferred_element_type=jnp.float32)
    qk_ref[...] = jnp.concatenate([q, k], axis=-1)


def _node_tc(nf, Wn, bn, Wsrc, Wdst):
    blk = 2000
    return pl.pallas_call(
        _node_body,
        grid=(N // blk,),
        in_specs=[
            pl.BlockSpec((blk, D), lambda i: (i, 0)),
            pl.BlockSpec((D, R), lambda i: (0, 0)),
            pl.BlockSpec((1, R), lambda i: (0, 0)),
            pl.BlockSpec((R, P), lambda i: (0, 0)),
            pl.BlockSpec((R, P), lambda i: (0, 0)),
        ],
        out_specs=pl.BlockSpec((blk, 2 * P), lambda i: (i, 0)),
        out_shape=jax.ShapeDtypeStruct((N, 2 * P), jnp.float32),
    )(nf, Wn, bn.reshape(1, R), Wsrc, Wdst)


def _sc_gather_body(tbl_hbm, src2_hbm, dst2_hbm, qskd_hbm,
                    sp_tbl, sidx_all, didx_all,
                    qs_a, kd_a, qs_b, kd_b,
                    tqs_v, tkd_v,
                    sg_a, sg_b, sw_a, sw_b, sem_t):
    sid = lax.axis_index("s")
    wid = sid * _NC + lax.axis_index("c")
    base_w = pl.multiple_of(wid * _EPW, 8)

    trows = (2 * N) // _NS
    trow0 = pl.multiple_of(sid * trows, 8)
    pltpu.sync_copy(tbl_hbm.at[pl.ds(trow0, trows)],
                    sp_tbl.at[pl.ds(trow0, trows)])
    pltpu.sync_copy(src2_hbm.at[pl.ds(base_w, _EPW)], sidx_all)
    pltpu.sync_copy(dst2_hbm.at[pl.ds(base_w, _EPW)], didx_all)
    plsc.subcore_barrier()

    def issue_gather(c, qs, kd, sem):
        off = pl.multiple_of(c * _C, 8)
        cq = pltpu.async_copy(sp_tbl.at[sidx_all.at[pl.ds(off, _C)]], qs, sem)
        ck = pltpu.async_copy(sp_tbl.at[didx_all.at[pl.ds(off, _C)]], kd, sem)
        return cq, ck

    def issue_writeback(c, qs, kd, sem):
        base = pl.multiple_of(base_w + c * _C, 8)
        wq = pltpu.async_copy(
            qs, qskd_hbm.at[pl.ds(base, _C), pl.ds(0, P)], sem)
        wk = pltpu.async_copy(
            kd, qskd_hbm.at[pl.ds(base, _C), pl.ds(P, P)], sem)
        return wq, wk

    def pair_body(i, _):
        c0 = 2 * i
        ga = issue_gather(c0, qs_a, kd_a, sg_a)
        gb = issue_gather(c0 + 1, qs_b, kd_b, sg_b)
        ga[0].wait()
        ga[1].wait()
        wa = issue_writeback(c0, qs_a, kd_a, sw_a)
        gb[0].wait()
        gb[1].wait()
        wb = issue_writeback(c0 + 1, qs_b, kd_b, sw_b)
        wa[0].wait()
        wa[1].wait()
        wb[0].wait()
        wb[1].wait()
        return 0

    lax.fori_loop(0, _NFULL // 2, pair_body, 0)

    ga = issue_gather(_NFULL - 1, qs_a, kd_a, sg_a)
    toff = pl.multiple_of(_NFULL * _C, 8)
    tbase = pl.multiple_of(base_w + _NFULL * _C, 8)
    cq = pltpu.async_copy(sp_tbl.at[sidx_all.at[pl.ds(toff, _CT)]],
                          tqs_v, sem_t)
    ck = pltpu.async_copy(sp_tbl.at[didx_all.at[pl.ds(toff, _CT)]],
                          tkd_v, sem_t)
    ga[0].wait()
    ga[1].wait()
    wa = issue_writeback(_NFULL - 1, qs_a, kd_a, sw_a)
    cq.wait()
    ck.wait()
    pltpu.sync_copy(tqs_v, qskd_hbm.at[pl.ds(tbase, _CT), pl.ds(0, P)])
    pltpu.sync_copy(tkd_v, qskd_hbm.at[pl.ds(tbase, _CT), pl.ds(P, P)])
    wa[0].wait()
    wa[1].wait()


def _sc_gather(qk_tbl, src2, dst2):
    mesh = plsc.VectorSubcoreMesh(core_axis_name="c", subcore_axis_name="s")
    kern = functools.partial(
        pl.kernel,
        out_type=jax.ShapeDtypeStruct((E, 2 * P), jnp.float32),
        mesh=mesh,
        scratch_types=[
            pltpu.VMEM_SHARED((2 * N, P), jnp.float32),
            pltpu.VMEM((_EPW,), jnp.int32),
            pltpu.VMEM((_EPW,), jnp.int32),
            pltpu.VMEM((_C, P), jnp.float32),
            pltpu.VMEM((_C, P), jnp.float32),
            pltpu.VMEM((_C, P), jnp.float32),
            pltpu.VMEM((_C, P), jnp.float32),
            pltpu.VMEM((_CT, P), jnp.float32),
            pltpu.VMEM((_CT, P), jnp.float32),
            pltpu.SemaphoreType.DMA,
            pltpu.SemaphoreType.DMA,
            pltpu.SemaphoreType.DMA,
            pltpu.SemaphoreType.DMA,
            pltpu.SemaphoreType.DMA,
        ],
        compiler_params=pltpu.CompilerParams(use_tc_tiling_on_sc=False),
    )(_sc_gather_body)
    return kern(qk_tbl, src2, dst2)


_EBLK = 3200


def _edge_body(ef_ref, we_ref, be_ref, wedge_ref, qskd_ref, score_ref):
    e = jnp.maximum(
        jnp.dot(ef_ref[...], we_ref[...], preferred_element_type=jnp.float32)
        + be_ref[...], 0.0)
    ep = jnp.dot(e, wedge_ref[...], preferred_element_type=jnp.float32)
    qs = qskd_ref[:, :P]
    kd = qskd_ref[:, P:]
    s = jnp.sum(qs * kd + ep * (qs + kd), axis=-1)
    score_ref[...] = s.reshape(1, _EBLK // 128, 128)


def _edge_tc(ef, We, be, Wedge, qskd):
    out = pl.pallas_call(
        _edge_body,
        grid=(E // _EBLK,),
        in_specs=[
            pl.BlockSpec((_EBLK, D), lambda i: (i, 0)),
            pl.BlockSpec((D, R), lambda i: (0, 0)),
            pl.BlockSpec((1, R), lambda i: (0, 0)),
            pl.BlockSpec((R, P), lambda i: (0, 0)),
            pl.BlockSpec((_EBLK, 2 * P), lambda i: (i, 0)),
        ],
        out_specs=pl.BlockSpec((1, _EBLK // 128, 128), lambda i: (i, 0, 0)),
        out_shape=jax.ShapeDtypeStruct(
            (E // _EBLK, _EBLK // 128, 128), jnp.float32),
    )(ef, We, be.reshape(1, R), Wedge, qskd)
    return out.reshape(E)


def kernel(node_features, edge_features, edge_index, Wn, bn, We, be,
           Wsrc, Wdst, Wedge):
    qk = _node_tc(node_features, Wn, bn, Wsrc, Wdst)
    qk_tbl = qk.reshape(2 * N, P)
    src2 = edge_index[0].astype(jnp.int32) * 2
    dst2 = edge_index[1].astype(jnp.int32) * 2 + 1
    qskd = _sc_gather(qk_tbl, src2, dst2)
    return _edge_tc(edge_features, We, be, Wedge, qskd)


# speedup vs baseline: 1.9198x; 1.0021x over previous
import functools

import jax
import jax.numpy as jnp
from jax import lax
from jax.experimental import pallas as pl
from jax.experimental.pallas import tpu as pltpu
from jax.experimental.pallas import tpu_sc as plsc

N = 10000
E = 160000
D = 256
R = 256
P = 64

_NC = 2
_NS = 16
_NW = _NC * _NS
_EPW = E // _NW
_C = 256
_NFULL = _EPW // _C
_CT = _EPW - _NFULL * _C


def _node_body(nf_ref, wn_ref, bn_ref, wsrc_ref, wdst_ref, qk_ref):
    n = jnp.maximum(
        jnp.dot(nf_ref[...], wn_ref[...], preferred_element_type=jnp.float32)
        + bn_ref[...], 0.0)
    q = jnp.dot(n, wsrc_ref[...], preferred_element_type=jnp.float32)
    k = jnp.dot(n, wdst_ref[...], preferred_element_type=jnp.float32)
    qk_ref[...] = jnp.concatenate([q, k], axis=-1)


def _node_tc(nf, Wn, bn, Wsrc, Wdst):
    blk = 2000
    return pl.pallas_call(
        _node_body,
        grid=(N // blk,),
        in_specs=[
            pl.BlockSpec((blk, D), lambda i: (i, 0)),
            pl.BlockSpec((D, R), lambda i: (0, 0)),
            pl.BlockSpec((1, R), lambda i: (0, 0)),
            pl.BlockSpec((R, P), lambda i: (0, 0)),
            pl.BlockSpec((R, P), lambda i: (0, 0)),
        ],
        out_specs=pl.BlockSpec((blk, 2 * P), lambda i: (i, 0)),
        out_shape=jax.ShapeDtypeStruct((N, 2 * P), jnp.float32),
    )(nf, Wn, bn.reshape(1, R), Wsrc, Wdst)


def _sc_gather_body(tbl_hbm, src2_hbm, dst2_hbm, qskd_hbm,
                    sidx_all, didx_all,
                    qs_a, kd_a, qs_b, kd_b,
                    tqs_v, tkd_v,
                    sg_a, sg_b, sw_a, sw_b, sem_t):
    wid = lax.axis_index("s") * _NC + lax.axis_index("c")
    base_w = pl.multiple_of(wid * _EPW, 8)

    pltpu.sync_copy(src2_hbm.at[pl.ds(base_w, _EPW)], sidx_all)
    pltpu.sync_copy(dst2_hbm.at[pl.ds(base_w, _EPW)], didx_all)

    def issue_gather(c, qs, kd, sem):
        off = pl.multiple_of(c * _C, 8)
        cq = pltpu.async_copy(tbl_hbm.at[sidx_all.at[pl.ds(off, _C)]], qs, sem)
        ck = pltpu.async_copy(tbl_hbm.at[didx_all.at[pl.ds(off, _C)]], kd, sem)
        return cq, ck

    def issue_writeback(c, qs, kd, sem):
        base = pl.multiple_of(base_w + c * _C, 8)
        wq = pltpu.async_copy(
            qs, qskd_hbm.at[pl.ds(base, _C), pl.ds(0, P)], sem)
        wk = pltpu.async_copy(
            kd, qskd_hbm.at[pl.ds(base, _C), pl.ds(P, P)], sem)
        return wq, wk

    def pair_body(i, _):
        c0 = 2 * i
        ga = issue_gather(c0, qs_a, kd_a, sg_a)
        gb = issue_gather(c0 + 1, qs_b, kd_b, sg_b)
        ga[0].wait()
        ga[1].wait()
        wa = issue_writeback(c0, qs_a, kd_a, sw_a)
        gb[0].wait()
        gb[1].wait()
        wb = issue_writeback(c0 + 1, qs_b, kd_b, sw_b)
        wa[0].wait()
        wa[1].wait()
        wb[0].wait()
        wb[1].wait()
        return 0

    lax.fori_loop(0, _NFULL // 2, pair_body, 0)

    ga = issue_gather(_NFULL - 1, qs_a, kd_a, sg_a)
    toff = pl.multiple_of(_NFULL * _C, 8)
    tbase = pl.multiple_of(base_w + _NFULL * _C, 8)
    cq = pltpu.async_copy(tbl_hbm.at[sidx_all.at[pl.ds(toff, _CT)]],
                          tqs_v, sem_t)
    ck = pltpu.async_copy(tbl_hbm.at[didx_all.at[pl.ds(toff, _CT)]],
                          tkd_v, sem_t)
    ga[0].wait()
    ga[1].wait()
    wa = issue_writeback(_NFULL - 1, qs_a, kd_a, sw_a)
    cq.wait()
    ck.wait()
    pltpu.sync_copy(tqs_v, qskd_hbm.at[pl.ds(tbase, _CT), pl.ds(0, P)])
    pltpu.sync_copy(tkd_v, qskd_hbm.at[pl.ds(tbase, _CT), pl.ds(P, P)])
    wa[0].wait()
    wa[1].wait()


def _sc_gather(qk_tbl, src2, dst2):
    mesh = plsc.VectorSubcoreMesh(core_axis_name="c", subcore_axis_name="s")
    kern = functools.partial(
        pl.kernel,
        out_type=jax.ShapeDtypeStruct((E, 2 * P), jnp.float32),
        mesh=mesh,
        scratch_types=[
            pltpu.VMEM((_EPW,), jnp.int32),
            pltpu.VMEM((_EPW,), jnp.int32),
            pltpu.VMEM((_C, P), jnp.float32),
            pltpu.VMEM((_C, P), jnp.float32),
            pltpu.VMEM((_C, P), jnp.float32),
            pltpu.VMEM((_C, P), jnp.float32),
            pltpu.VMEM((_CT, P), jnp.float32),
            pltpu.VMEM((_CT, P), jnp.float32),
            pltpu.SemaphoreType.DMA,
            pltpu.SemaphoreType.DMA,
            pltpu.SemaphoreType.DMA,
            pltpu.SemaphoreType.DMA,
            pltpu.SemaphoreType.DMA,
        ],
        compiler_params=pltpu.CompilerParams(use_tc_tiling_on_sc=False),
    )(_sc_gather_body)
    return kern(qk_tbl, src2, dst2)


_EBLK = 3200


def _edge_body(ef_ref, we_ref, be_ref, wedge_ref, qskd_ref, score_ref):
    e = jnp.maximum(
        jnp.dot(ef_ref[...], we_ref[...], preferred_element_type=jnp.float32)
        + be_ref[...], 0.0)
    ep = jnp.dot(e, wedge_ref[...], preferred_element_type=jnp.float32)
    qs = qskd_ref[:, :P]
    kd = qskd_ref[:, P:]
    s = jnp.sum(qs * kd + ep * (qs + kd), axis=-1)
    score_ref[...] = s.reshape(1, _EBLK // 128, 128)


def _edge_tc(ef, We, be, Wedge, qskd):
    out = pl.pallas_call(
        _edge_body,
        grid=(E // _EBLK,),
        in_specs=[
            pl.BlockSpec((_EBLK, D), lambda i: (i, 0)),
            pl.BlockSpec((D, R), lambda i: (0, 0)),
            pl.BlockSpec((1, R), lambda i: (0, 0)),
            pl.BlockSpec((R, P), lambda i: (0, 0)),
            pl.BlockSpec((_EBLK, 2 * P), lambda i: (i, 0)),
        ],
        out_specs=pl.BlockSpec((1, _EBLK // 128, 128), lambda i: (i, 0, 0)),
        out_shape=jax.ShapeDtypeStruct(
            (E // _EBLK, _EBLK // 128, 128), jnp.float32),
    )(ef, We, be.reshape(1, R), Wedge, qskd)
    return out.reshape(E)


def kernel(node_features, edge_features, edge_index, Wn, bn, We, be,
           Wsrc, Wdst, Wedge):
    qk = _node_tc(node_features, Wn, bn, Wsrc, Wdst)
    qk_tbl = qk.reshape(2 * N, P)
    src2 = edge_index[0].astype(jnp.int32) * 2
    dst2 = edge_index[1].astype(jnp.int32) * 2 + 1
    qskd = _sc_gather(qk_tbl, src2, dst2)
    return _edge_tc(edge_features, We, be, Wedge, qskd)
